# baseline (speedup 1.0000x reference)
import functools

import jax
import jax.numpy as jnp
from jax import lax
from jax.experimental import pallas as pl
from jax.experimental.pallas import tpu as pltpu
from jax.experimental.pallas import tpu_sc as plsc

N_NODES = 100000
N_EDGES = 6400000
NC = 2
NS = 16
NW = NC * NS
NP = 100352
SL = NP // NS
QSL = SL // 4
CHUNK1 = 10000
NITER1 = N_EDGES // (CHUNK1 * NW)
CHUNK = 2000
NITER = N_EDGES // (CHUNK * NW)

_MESH = plsc.VectorSubcoreMesh(
    core_axis_name="c", subcore_axis_name="s", num_cores=NC, num_subcores=NS
)


def _worker_id():
    c = lax.axis_index("c")
    s = lax.axis_index("s")
    return c, s, s * NC + c


@functools.partial(
    pl.kernel,
    out_type=jax.ShapeDtypeStruct((NC * NP,), jnp.float32),
    mesh=_MESH,
    compiler_params=pltpu.CompilerParams(needs_layout_passes=False),
    scratch_types=[
        pltpu.VMEM((CHUNK1,), jnp.int32),
        pltpu.VMEM((CHUNK1,), jnp.int32),
        pltpu.VMEM((CHUNK1,), jnp.float32),
        pltpu.VMEM_SHARED((NP,), jnp.float32),
        pltpu.SemaphoreType.DMA,
        pltpu.SemaphoreType.DMA,
        pltpu.SemaphoreType.DMA,
        pltpu.SemaphoreType.DMA,
    ],
)
def _k1_degree(dst_hbm, zeros_hbm, ones_hbm, deg_out,
               dst0, dst1, ones_v, degacc, in0, in1, sc0, sc1):
    c, s, w = _worker_id()
    dst = (dst0, dst1)
    in_sem = (in0, in1)
    sc_sem = (sc0, sc1)

    @pl.when(s == 0)
    def _():
        pltpu.sync_copy(zeros_hbm, degacc)

    pltpu.sync_copy(ones_hbm, ones_v)
    plsc.subcore_barrier()

    def _in(k, b):
        pltpu.async_copy(
            dst_hbm.at[pl.ds((w + NW * k) * CHUNK1, CHUNK1)], dst[b], in_sem[b]
        )

    def _wait_in(b):
        pltpu.make_async_copy(
            dst_hbm.at[pl.ds(0, CHUNK1)], dst[b], in_sem[b]
        ).wait()

    def _wait_sc(b):
        pltpu.make_async_copy(ones_v, degacc.at[dst[b]], sc_sem[b]).wait()

    _in(0, 0)

    def body(i, carry):
        for b in (0, 1):
            k = 2 * i + b
            _wait_in(b)
            pltpu.async_copy(ones_v, degacc.at[dst[b]], sc_sem[b], add=True)
            if b == 0:
                @pl.when(i > 0)
                def _():
                    _wait_sc(1)
                _in(k + 1, 1)
            else:
                _wait_sc(0)

                @pl.when(i < NITER1 // 2 - 1)
                def _():
                    _in(k + 1, 0)

        return carry

    lax.fori_loop(0, NITER1 // 2, body, None)
    _wait_sc(1)
    plsc.subcore_barrier()

    @pl.when(s == 0)
    def _():
        pltpu.sync_copy(degacc, deg_out.at[pl.ds(c * NP, NP)])


def _rsqrt16(v):
    half = 0.5 * v
    i = plsc.bitcast(v, jnp.int32)
    i = 0x5F3759DF - lax.shift_right_logical(i, 1)
    r = plsc.bitcast(i, jnp.float32)
    for _ in range(3):
        r = r * (1.5 - half * r * r)
    return r


@functools.partial(
    pl.kernel,
    out_type=[
        jax.ShapeDtypeStruct((NC * NP,), jnp.float32),
        jax.ShapeDtypeStruct((NC * NP,), jnp.float32),
    ],
    mesh=_MESH,
    compiler_params=pltpu.CompilerParams(needs_layout_passes=False),
    scratch_types=[
        pltpu.VMEM((NP,), jnp.float32),
        pltpu.VMEM((CHUNK,), jnp.int32),
        pltpu.VMEM((CHUNK,), jnp.int32),
        pltpu.VMEM((CHUNK,), jnp.int32),
        pltpu.VMEM((CHUNK,), jnp.int32),
        pltpu.VMEM((CHUNK,), jnp.float32),
        pltpu.VMEM((CHUNK,), jnp.float32),
        pltpu.VMEM_SHARED((NP,), jnp.float32),
        pltpu.SemaphoreType.DMA,
        pltpu.SemaphoreType.DMA,
        pltpu.SemaphoreType.DMA,
        pltpu.SemaphoreType.DMA,
    ],
)
def _k3_aggregate(src_hbm, dst_hbm, degp_hbm, x_hbm, zeros_hbm,
                  acc_out, yscr,
                  ybuf, src0, src1, dst0, dst1, msg0, msg1, sacc,
                  in0, in1, sc0, sc1):
    c, s, w = _worker_id()
    src = (src0, src1)
    dst = (dst0, dst1)
    msg = (msg0, msg1)
    in_sem = (in0, in1)
    sc_sem = (sc0, sc1)

    def _in(k, b):
        base = (w + NW * k) * CHUNK
        pltpu.async_copy(src_hbm.at[pl.ds(base, CHUNK)], src[b], in_sem[b])
        pltpu.async_copy(dst_hbm.at[pl.ds(base, CHUNK)], dst[b], in_sem[b])

    _in(0, 0)

    @pl.when(s == 0)
    def _():
        pltpu.sync_copy(zeros_hbm, sacc)

    base = s * SL
    for quarter in range(4):
        hbase = base + quarter * QSL
        pltpu.sync_copy(degp_hbm.at[pl.ds(hbase, QSL)], msg0.at[pl.ds(0, QSL)])
        pltpu.sync_copy(
            degp_hbm.at[pl.ds(NP + hbase, QSL)], msg1.at[pl.ds(0, QSL)]
        )
        pltpu.sync_copy(x_hbm.at[pl.ds(hbase, QSL)], src1.at[pl.ds(0, QSL)])
        def _ybody(t, carry):
            sl16 = pl.ds(t * 16, 16)
            deg = msg0[sl16] + msg1[sl16] + 1.0
            xv = plsc.bitcast(src1[sl16], jnp.float32)
            ybuf[pl.ds(hbase + t * 16, 16)] = _rsqrt16(deg) * xv
            return carry

        lax.fori_loop(0, QSL // 16, _ybody, None)
        pltpu.sync_copy(
            ybuf.at[pl.ds(hbase, QSL)], yscr.at[pl.ds(c * NP + hbase, QSL)]
        )
    plsc.subcore_barrier()
    pltpu.sync_copy(yscr.at[pl.ds(c * NP, NP)], ybuf)

    def _wait_in(b):
        pltpu.make_async_copy(
            src_hbm.at[pl.ds(0, CHUNK)], src[b], in_sem[b]
        ).wait()
        pltpu.make_async_copy(
            dst_hbm.at[pl.ds(0, CHUNK)], dst[b], in_sem[b]
        ).wait()

    def _wait_sc(b):
        pltpu.make_async_copy(msg[b], sacc.at[dst[b]], sc_sem[b]).wait()

    def body(i, carry):
        for b in (0, 1):
            k = 2 * i + b
            _wait_in(b)
            for t in range(CHUNK // 16):
                idx16 = src[b][pl.ds(t * 16, 16)]
                msg[b][pl.ds(t * 16, 16)] = plsc.load_gather(ybuf, [idx16])
            pltpu.async_copy(msg[b], sacc.at[dst[b]], sc_sem[b], add=True)
            if b == 0:
                @pl.when(i > 0)
                def _():
                    _wait_sc(1)
                _in(k + 1, 1)
            else:
                _wait_sc(0)

                @pl.when(i < NITER // 2 - 1)
                def _():
                    _in(k + 1, 0)

        return carry

    lax.fori_loop(0, NITER // 2, body, None)
    _wait_sc(1)
    plsc.subcore_barrier()

    @pl.when(s == 0)
    def _():
        pltpu.sync_copy(sacc, acc_out.at[pl.ds(c * NP, NP)])


def _k4_body(dp_ref, ap_ref, y_ref, wb_ref, out_ref):
    dinv = lax.rsqrt(dp_ref[0] + dp_ref[1] + 1.0)
    out0 = dinv * (ap_ref[0] + ap_ref[1] + y_ref[...])
    out_ref[0] = out0 * wb_ref[0] + wb_ref[2]
    out_ref[1] = out0 * wb_ref[1] + wb_ref[3]


_R, _C = 784, 128


def kernel(x, edge_index, W, b):
    ei32 = edge_index.astype(jnp.int32)
    zeros = jnp.zeros((NP,), jnp.float32)
    ones = jnp.ones((CHUNK1,), jnp.float32)
    xpad = jnp.concatenate(
        [x.reshape(N_NODES), jnp.zeros((NP - N_NODES,), jnp.float32)]
    )
    xbits = lax.bitcast_convert_type(xpad, jnp.int32)

    deg_part = _k1_degree(ei32[1], zeros, ones)

    acc_part, y_scr = _k3_aggregate(ei32[0], ei32[1], deg_part, xbits, zeros)

    wb = jnp.concatenate([W[0], b]).astype(jnp.float32)
    out2 = pl.pallas_call(
        _k4_body,
        out_shape=jax.ShapeDtypeStruct((NC, _R, _C), jnp.float32),
        in_specs=[
            pl.BlockSpec(memory_space=pltpu.VMEM),
            pl.BlockSpec(memory_space=pltpu.VMEM),
            pl.BlockSpec(memory_space=pltpu.VMEM),
            pl.BlockSpec(memory_space=pltpu.SMEM),
        ],
        out_specs=pl.BlockSpec(memory_space=pltpu.VMEM),
    )(
        deg_part.reshape(NC, _R, _C),
        acc_part.reshape(NC, _R, _C),
        y_scr.reshape(NC, _R, _C)[0],
        wb,
    )

    return out2.reshape(NC, NP)[:, :N_NODES].T


# speedup vs baseline: 1.0944x; 1.0944x over previous
import functools

import jax
import jax.numpy as jnp
from jax import lax
from jax.experimental import pallas as pl
from jax.experimental.pallas import tpu as pltpu
from jax.experimental.pallas import tpu_sc as plsc

N_NODES = 100000
N_EDGES = 6400000
NC = 2
NS = 16
NW = NC * NS
NP = 100352
SL = NP // NS
QSL = SL // 4
CHUNK1 = 10000
NITER1 = N_EDGES // (CHUNK1 * NW)
CHUNK = 4000
NITER = N_EDGES // (CHUNK * NW)
YB = 100096

_MESH = plsc.VectorSubcoreMesh(
    core_axis_name="c", subcore_axis_name="s", num_cores=NC, num_subcores=NS
)


def _worker_id():
    c = lax.axis_index("c")
    s = lax.axis_index("s")
    return c, s, s * NC + c


@functools.partial(
    pl.kernel,
    out_type=jax.ShapeDtypeStruct((NC * NP,), jnp.float32),
    mesh=_MESH,
    compiler_params=pltpu.CompilerParams(needs_layout_passes=False),
    scratch_types=[
        pltpu.VMEM((CHUNK1,), jnp.int32),
        pltpu.VMEM((CHUNK1,), jnp.int32),
        pltpu.VMEM((CHUNK1,), jnp.float32),
        pltpu.VMEM_SHARED((NP,), jnp.float32),
        pltpu.SemaphoreType.DMA,
        pltpu.SemaphoreType.DMA,
        pltpu.SemaphoreType.DMA,
        pltpu.SemaphoreType.DMA,
    ],
)
def _k1_degree(dst_hbm, zeros_hbm, ones_hbm, deg_out,
               dst0, dst1, ones_v, degacc, in0, in1, sc0, sc1):
    c, s, w = _worker_id()
    dst = (dst0, dst1)
    in_sem = (in0, in1)
    sc_sem = (sc0, sc1)

    @pl.when(s == 0)
    def _():
        pltpu.sync_copy(zeros_hbm, degacc)

    pltpu.sync_copy(ones_hbm, ones_v)
    plsc.subcore_barrier()

    def _in(k, b):
        pltpu.async_copy(
            dst_hbm.at[pl.ds((w + NW * k) * CHUNK1, CHUNK1)], dst[b], in_sem[b]
        )

    def _wait_in(b):
        pltpu.make_async_copy(
            dst_hbm.at[pl.ds(0, CHUNK1)], dst[b], in_sem[b]
        ).wait()

    def _wait_sc(b):
        pltpu.make_async_copy(ones_v, degacc.at[dst[b]], sc_sem[b]).wait()

    _in(0, 0)

    def body(i, carry):
        for b in (0, 1):
            k = 2 * i + b
            _wait_in(b)
            pltpu.async_copy(ones_v, degacc.at[dst[b]], sc_sem[b], add=True)
            if b == 0:
                @pl.when(i > 0)
                def _():
                    _wait_sc(1)
                _in(k + 1, 1)
            else:
                _wait_sc(0)

                @pl.when(i < NITER1 // 2 - 1)
                def _():
                    _in(k + 1, 0)

        return carry

    lax.fori_loop(0, NITER1 // 2, body, None)
    _wait_sc(1)
    plsc.subcore_barrier()

    @pl.when(s == 0)
    def _():
        pltpu.sync_copy(degacc, deg_out.at[pl.ds(c * NP, NP)])


def _rsqrt16(v):
    half = 0.5 * v
    i = plsc.bitcast(v, jnp.int32)
    i = 0x5F3759DF - lax.shift_right_logical(i, 1)
    r = plsc.bitcast(i, jnp.float32)
    for _ in range(3):
        r = r * (1.5 - half * r * r)
    return r


@functools.partial(
    pl.kernel,
    out_type=[
        jax.ShapeDtypeStruct((NC * NP,), jnp.float32),
        jax.ShapeDtypeStruct((NC * NP,), jnp.float32),
    ],
    mesh=_MESH,
    compiler_params=pltpu.CompilerParams(needs_layout_passes=False),
    scratch_types=[
        pltpu.VMEM((YB,), jnp.float32),
        pltpu.VMEM((CHUNK,), jnp.int32),
        pltpu.VMEM((CHUNK,), jnp.int32),
        pltpu.VMEM((CHUNK,), jnp.int32),
        pltpu.VMEM((CHUNK,), jnp.int32),
        pltpu.VMEM((CHUNK,), jnp.float32),
        pltpu.VMEM((CHUNK,), jnp.float32),
        pltpu.VMEM_SHARED((YB,), jnp.float32),
        pltpu.SemaphoreType.DMA,
        pltpu.SemaphoreType.DMA,
        pltpu.SemaphoreType.DMA,
        pltpu.SemaphoreType.DMA,
    ],
)
def _k3_aggregate(src_hbm, dst_hbm, degp_hbm, x_hbm, zeros_hbm,
                  acc_out, yscr,
                  ybuf, src0, src1, dst0, dst1, msg0, msg1, sacc,
                  in0, in1, sc0, sc1):
    c, s, w = _worker_id()
    src = (src0, src1)
    dst = (dst0, dst1)
    msg = (msg0, msg1)
    in_sem = (in0, in1)
    sc_sem = (sc0, sc1)

    def _in(k, b):
        base = (w + NW * k) * CHUNK
        pltpu.async_copy(src_hbm.at[pl.ds(base, CHUNK)], src[b], in_sem[b])
        pltpu.async_copy(dst_hbm.at[pl.ds(base, CHUNK)], dst[b], in_sem[b])

    _in(0, 0)

    @pl.when(s == 0)
    def _():
        pltpu.sync_copy(zeros_hbm.at[pl.ds(0, YB)], sacc)

    base = s * SL
    for quarter in range(4):
        hbase = base + quarter * QSL
        pltpu.sync_copy(degp_hbm.at[pl.ds(hbase, QSL)], msg0.at[pl.ds(0, QSL)])
        pltpu.sync_copy(
            degp_hbm.at[pl.ds(NP + hbase, QSL)], msg1.at[pl.ds(0, QSL)]
        )
        pltpu.sync_copy(x_hbm.at[pl.ds(hbase, QSL)], src1.at[pl.ds(0, QSL)])
        def _ybody(t, carry):
            sl16 = pl.ds(t * 16, 16)
            deg = msg0[sl16] + msg1[sl16] + 1.0
            xv = plsc.bitcast(src1[sl16], jnp.float32)
            msg0[sl16] = _rsqrt16(deg) * xv
            return carry

        lax.fori_loop(0, QSL // 16, _ybody, None)
        pltpu.sync_copy(
            msg0.at[pl.ds(0, QSL)], yscr.at[pl.ds(c * NP + hbase, QSL)]
        )
    plsc.subcore_barrier()
    pltpu.sync_copy(yscr.at[pl.ds(c * NP, YB)], ybuf)

    def _wait_in(b):
        pltpu.make_async_copy(
            src_hbm.at[pl.ds(0, CHUNK)], src[b], in_sem[b]
        ).wait()
        pltpu.make_async_copy(
            dst_hbm.at[pl.ds(0, CHUNK)], dst[b], in_sem[b]
        ).wait()

    def _wait_sc(b):
        pltpu.make_async_copy(msg[b], sacc.at[dst[b]], sc_sem[b]).wait()

    def body(i, carry):
        for b in (0, 1):
            k = 2 * i + b
            _wait_in(b)
            for t in range(CHUNK // 16):
                idx16 = src[b][pl.ds(t * 16, 16)]
                msg[b][pl.ds(t * 16, 16)] = plsc.load_gather(ybuf, [idx16])
            pltpu.async_copy(msg[b], sacc.at[dst[b]], sc_sem[b], add=True)
            if b == 0:
                @pl.when(i > 0)
                def _():
                    _wait_sc(1)
                _in(k + 1, 1)
            else:
                _wait_sc(0)

                @pl.when(i < NITER // 2 - 1)
                def _():
                    _in(k + 1, 0)

        return carry

    lax.fori_loop(0, NITER // 2, body, None)
    _wait_sc(1)
    plsc.subcore_barrier()

    @pl.when(s == 0)
    def _():
        pltpu.sync_copy(sacc, acc_out.at[pl.ds(c * NP, YB)])


def _k4_body(dp_ref, ap_ref, y_ref, wb_ref, out_ref):
    dinv = lax.rsqrt(dp_ref[0] + dp_ref[1] + 1.0)
    out0 = dinv * (ap_ref[0] + ap_ref[1] + y_ref[...])
    out_ref[0] = out0 * wb_ref[0] + wb_ref[2]
    out_ref[1] = out0 * wb_ref[1] + wb_ref[3]


_R, _C = 784, 128


def kernel(x, edge_index, W, b):
    ei32 = edge_index.astype(jnp.int32)
    zeros = jnp.zeros((NP,), jnp.float32)
    ones = jnp.ones((CHUNK1,), jnp.float32)
    xpad = jnp.concatenate(
        [x.reshape(N_NODES), jnp.zeros((NP - N_NODES,), jnp.float32)]
    )
    xbits = lax.bitcast_convert_type(xpad, jnp.int32)

    deg_part = _k1_degree(ei32[1], zeros, ones)

    acc_part, y_scr = _k3_aggregate(ei32[0], ei32[1], deg_part, xbits, zeros)

    wb = jnp.concatenate([W[0], b]).astype(jnp.float32)
    out2 = pl.pallas_call(
        _k4_body,
        out_shape=jax.ShapeDtypeStruct((NC, _R, _C), jnp.float32),
        in_specs=[
            pl.BlockSpec(memory_space=pltpu.VMEM),
            pl.BlockSpec(memory_space=pltpu.VMEM),
            pl.BlockSpec(memory_space=pltpu.VMEM),
            pl.BlockSpec(memory_space=pltpu.SMEM),
        ],
        out_specs=pl.BlockSpec(memory_space=pltpu.VMEM),
    )(
        deg_part.reshape(NC, _R, _C),
        acc_part.reshape(NC, _R, _C),
        y_scr.reshape(NC, _R, _C)[0],
        wb,
    )

    return out2.reshape(NC, NP)[:, :N_NODES].T
